# Initial kernel scaffold; baseline (speedup 1.0000x reference)
import jax
import jax.numpy as jnp
from jax.experimental import pallas as pl


def kernel(inputs, emb_table, wide_table, W0, b0, W1, b1, W2, b2, Wo, bo):
    raise NotImplementedError("write your pallas kernel here")


# speedup vs baseline: 4.5742x; 4.5742x over previous
import functools

import jax
import jax.numpy as jnp
from jax import lax
from jax.experimental import pallas as pl
from jax.experimental.pallas import tpu as pltpu
from jax.experimental.pallas import tpu_sc as plsc

_B = 4096
_F = 26
_V = 100000
_D = 32
_H = 512

_NC = 2
_NS = 16
_NW = _NC * _NS


def _make_gather(n_idx, d):
  per_w = n_idx // _NW
  mesh = plsc.VectorSubcoreMesh(core_axis_name="c", subcore_axis_name="s")

  @functools.partial(
      pl.kernel,
      out_type=(
          jax.ShapeDtypeStruct((n_idx, d), jnp.float32),
          jax.ShapeDtypeStruct((n_idx,), jnp.float32),
      ),
      mesh=mesh,
      compiler_params=pltpu.CompilerParams(use_tc_tiling_on_sc=False),
      scratch_types=[
          pltpu.VMEM((per_w,), jnp.int32),
          pltpu.VMEM((per_w, d), jnp.float32),
          pltpu.VMEM((per_w,), jnp.float32),
          pltpu.SemaphoreType.DMA,
          pltpu.SemaphoreType.DMA,
      ],
  )
  def gather(idx_hbm, emb_hbm, wide_hbm, emb_out, wide_out,
             idx_v, rows_v, wvals_v, sem, wsem):
    wid = lax.axis_index("s") * _NC + lax.axis_index("c")
    base = wid * per_w
    pltpu.sync_copy(idx_hbm.at[pl.ds(base, per_w)], idx_v)
    cp = pltpu.async_copy(emb_hbm.at[idx_v], rows_v, sem)
    wp = pltpu.async_copy(wide_hbm.at[idx_v], wvals_v, wsem)
    cp.wait()
    wp.wait()
    pltpu.sync_copy(rows_v, emb_out.at[pl.ds(base, per_w)])
    pltpu.sync_copy(wvals_v, wide_out.at[pl.ds(base, per_w)])

  return gather


def _mlp_body(x_ref, wv_ref, w0_ref, b0_ref, w1_ref, b1_ref, w2_ref, b2_ref,
              wo_ref, bo_ref, o_ref):
  x = x_ref[...]
  h = jnp.maximum(
      jnp.dot(x, w0_ref[...], preferred_element_type=jnp.float32)
      + b0_ref[...], 0.0)
  h = jnp.maximum(
      jnp.dot(h, w1_ref[...], preferred_element_type=jnp.float32)
      + b1_ref[...], 0.0)
  h = jnp.maximum(
      jnp.dot(h, w2_ref[...], preferred_element_type=jnp.float32)
      + b2_ref[...], 0.0)
  deep = jnp.dot(h, wo_ref[...], preferred_element_type=jnp.float32)
  wide = jnp.sum(wv_ref[...], axis=1, keepdims=True)
  logits = deep + wide + bo_ref[0, 0]
  o_ref[...] = 1.0 / (1.0 + jnp.exp(-logits))


def kernel(inputs, emb_table, wide_table, W0, b0, W1, b1, W2, b2, Wo, bo):
  bsz, f = inputs.shape
  v, d = emb_table.shape
  h = W0.shape[1]
  n_idx = bsz * f

  idx_flat = inputs.reshape(n_idx).astype(jnp.int32)
  wide_flat = wide_table.reshape(v)

  emb_flat, wvals_flat = _make_gather(n_idx, d)(idx_flat, emb_table, wide_flat)
  x = emb_flat.reshape(bsz, f * d)
  wv = wvals_flat.reshape(bsz, f)

  bb = 256
  grid = (bsz // bb,)
  out = pl.pallas_call(
      _mlp_body,
      grid=grid,
      in_specs=[
          pl.BlockSpec((bb, f * d), lambda i: (i, 0)),
          pl.BlockSpec((bb, f), lambda i: (i, 0)),
          pl.BlockSpec((f * d, h), lambda i: (0, 0)),
          pl.BlockSpec((1, h), lambda i: (0, 0)),
          pl.BlockSpec((h, h), lambda i: (0, 0)),
          pl.BlockSpec((1, h), lambda i: (0, 0)),
          pl.BlockSpec((h, h), lambda i: (0, 0)),
          pl.BlockSpec((1, h), lambda i: (0, 0)),
          pl.BlockSpec((h, 1), lambda i: (0, 0)),
          pl.BlockSpec((1, 1), lambda i: (0, 0)),
      ],
      out_specs=pl.BlockSpec((bb, 1), lambda i: (i, 0)),
      out_shape=jax.ShapeDtypeStruct((bsz, 1), jnp.float32),
  )(x, wv, W0, b0.reshape(1, h), W1, b1.reshape(1, h), W2, b2.reshape(1, h),
    Wo, bo.reshape(1, 1))
  return out
